# baseline (speedup 1.0000x reference)
import functools

import jax
import jax.numpy as jnp
from jax import lax
from jax.experimental import pallas as pl
from jax.experimental.pallas import tpu as pltpu
from jax.experimental.pallas import tpu_sc as plsc

NC, NS = 2, 16
NW = NC * NS

NBUF = 4
LOOK = 2
CHR = 240
IDX_BLK = 16384


def _fb_sc_call(f2, idx2, fb, idx_bank):
    N, F = f2.shape
    S = fb.shape[0]
    rest = S - N

    fper = N // NW
    per = rest // NW // 8 * 8
    left = rest - NW * per
    nch = per // CHR // NBUF * NBUF
    groups = nch // NBUF
    tail = per - nch * CHR
    assert 0 < tail <= CHR and tail % 8 == 0

    f_chunks = []
    off = 0
    while off < fper:
        sz = min(CHR, fper - off)
        f_chunks.append((off, sz))
        off += sz

    mesh = plsc.VectorSubcoreMesh(
        core_axis_name="c", subcore_axis_name="s",
        num_cores=NC, num_subcores=NS,
    )

    @functools.partial(
        pl.kernel,
        out_type=jax.ShapeDtypeStruct((S, F), fb.dtype),
        mesh=mesh,
        compiler_params=pltpu.CompilerParams(use_tc_tiling_on_sc=True),
        scratch_types=(
            [pltpu.VMEM((CHR, F), fb.dtype)] * NBUF
            + [pltpu.SemaphoreType.DMA((NBUF,)),
               pltpu.SemaphoreType.DMA((NBUF,))]
        ),
    )
    def k(f_h, fb_h, out_h, buf0, buf1, buf2, buf3, gsem, ssem):
        bufs = (buf0, buf1, buf2, buf3)
        c_ax = lax.axis_index("c")
        s_ax = lax.axis_index("s")
        w = s_ax * NC + c_ax
        fb_base = w * per
        ob_base = N + w * per

        for off, sz in f_chunks:
            r0 = w * fper + off
            pltpu.sync_copy(f_h.at[pl.ds(r0, sz)], buf0.at[pl.ds(0, sz)])
            pltpu.sync_copy(buf0.at[pl.ds(0, sz)], out_h.at[pl.ds(r0, sz)])

        def g_copy(c, b):
            return pltpu.make_async_copy(
                fb_h.at[pl.ds(fb_base + c * CHR, CHR)], bufs[b], gsem.at[b])

        def s_copy(c, b):
            return pltpu.make_async_copy(
                bufs[b], out_h.at[pl.ds(ob_base + c * CHR, CHR)], ssem.at[b])

        def g_tail(b):
            return pltpu.make_async_copy(
                fb_h.at[pl.ds(fb_base + nch * CHR, tail)],
                bufs[b].at[pl.ds(0, tail)], gsem.at[b])

        g_copy(0, 0).start()
        g_copy(1, 1).start()

        def group_body(g, carry):
            for b in range(NBUF):
                c = NBUF * g + b
                g_copy(c, b).wait()
                s_copy(c, b).start()
                j = c + LOOK
                bj = (b + LOOK) % NBUF

                @pl.when(j < nch)
                def _():
                    @pl.when(j >= NBUF)
                    def _():
                        s_copy(j - NBUF, bj).wait()
                    g_copy(j, bj).start()

                @pl.when(j == nch)
                def _():
                    s_copy(j - NBUF, bj).wait()
                    g_tail(bj).start()
            return carry

        lax.fori_loop(0, groups, group_body, 0)

        b_t = nch % NBUF
        g_tail(b_t).wait()
        pltpu.make_async_copy(
            bufs[b_t].at[pl.ds(0, tail)],
            out_h.at[pl.ds(ob_base + nch * CHR, tail)],
            ssem.at[b_t]).start()
        for c in range(nch - 3, nch):
            s_copy(c, c % NBUF).wait()
        pltpu.make_async_copy(
            bufs[b_t].at[pl.ds(0, tail)],
            out_h.at[pl.ds(ob_base + nch * CHR, tail)],
            ssem.at[b_t]).wait()

        @pl.when(w == 0)
        def _():
            pltpu.sync_copy(fb_h.at[pl.ds(NW * per, left)],
                            buf0.at[pl.ds(0, left)])
            pltpu.sync_copy(buf0.at[pl.ds(0, left)],
                            out_h.at[pl.ds(N + NW * per, left)])

    return k(f2, fb)


def _idx_body(idx_ref, idxb_ref, idxo_ref):
    i = pl.program_id(0)

    @pl.when(i == 0)
    def _():
        idxo_ref[...] = idx_ref[...]

    @pl.when(i > 0)
    def _():
        idxo_ref[...] = idxb_ref[...]


def _idx_tc_call(idx2, idx_bank):
    (N,) = idx2.shape
    (S,) = idx_bank.shape
    assert N == IDX_BLK
    nidx = pl.cdiv(S, IDX_BLK)
    return pl.pallas_call(
        _idx_body,
        grid=(nidx,),
        in_specs=[
            pl.BlockSpec((IDX_BLK,), lambda i: (0,)),
            pl.BlockSpec((IDX_BLK,), lambda i: (jnp.maximum(i - 1, 0),)),
        ],
        out_specs=pl.BlockSpec((IDX_BLK,), lambda i: (i,)),
        out_shape=jax.ShapeDtypeStruct((S,), idx_bank.dtype),
    )(idx2, idx_bank)


def kernel(f, idx, fb, idx_bank):
    f2 = f.reshape(-1, f.shape[-1])
    idx2 = idx.reshape(-1)
    out_fb = _fb_sc_call(f2, idx2, fb, idx_bank)
    out_idx = _idx_tc_call(idx2, idx_bank)
    return (out_fb, out_idx)
